# baseline (speedup 1.0000x reference)
import functools

import jax
import jax.numpy as jnp
from jax.experimental import pallas as pl
from jax.experimental.pallas import tpu as pltpu

_K = 8
_NB = 4
_NEG = -1e30
_VBLK = 256


def _knn1_body(vf_ref, pf_ref, ptsT_ref, vb_ref, pb_ref, vtx_ref, out_ref):
    vf = vf_ref[...]
    pf = pf_ref[...]
    rows, p = vf.shape[0], pf.shape[0]

    nsq = jnp.sum(pf * pf, axis=1, keepdims=True)
    inv = 1.0 / (jnp.sqrt(nsq) + 1e-12)
    pfn = pf * inv

    qsq = jnp.sum(vf * vf, axis=1, keepdims=True)
    qn = vf * (1.0 / (jnp.sqrt(qsq) + 1e-12))

    dn = (((1,), (1,)), ((), ()))
    S = jax.lax.dot_general(vf, pf, dn, precision=jax.lax.Precision.HIGHEST,
                            preferred_element_type=jnp.float32)
    C = jax.lax.dot_general(qn, pfn, dn,
                            preferred_element_type=jnp.float32)

    mask = vb_ref[...] == pb_ref[...]
    C = jnp.where(mask, C, _NEG)

    iota = jax.lax.broadcasted_iota(jnp.int32, (rows, p), 1)
    vtx = vtx_ref[...]

    for _ in range(_K):
        m = jnp.max(C, axis=1, keepdims=True)
        idx = jnp.min(jnp.where(C == m, iota, p), axis=1, keepdims=True)
        C = jnp.where(iota == idx, -jnp.inf, C)

    W = jnp.where(C == -jnp.inf, S, 0.0)
    den = jnp.sum(W, axis=1, keepdims=True)
    num0 = jnp.sum(W * ptsT_ref[0:1, :], axis=1, keepdims=True)
    num1 = jnp.sum(W * ptsT_ref[1:2, :], axis=1, keepdims=True)
    num2 = jnp.sum(W * ptsT_ref[2:3, :], axis=1, keepdims=True)

    out_ref[:, 0:1] = num0 / den - vtx[:, 0:1]
    out_ref[:, 1:2] = num1 / den - vtx[:, 1:2]
    out_ref[:, 2:3] = num2 / den - vtx[:, 2:3]


def _pv_from(logits, batch, mx, mn):
    s = jax.nn.sigmoid(logits)
    out = jnp.zeros_like(s)
    for b in range(_NB):
        out = jnp.where(batch == b, (s - mn[b]) / (mx[b] - mn[b]), out)
    return out


def _knn2_body(vf_ref, vfull_ref, vb_ref, vbr_ref, vl_ref, vlr_ref,
               flow_ref, flowT_ref, out_ref):
    vf = vf_ref[...]
    vfull = vfull_ref[...]
    rows, v = vf.shape[0], vfull.shape[0]

    vbr = vbr_ref[...]
    sr = jax.nn.sigmoid(vlr_ref[...])
    mx, mn = [], []
    for b in range(_NB):
        inb = vbr == b
        mx.append(jnp.max(jnp.where(inb, sr, -jnp.inf), axis=1, keepdims=True))
        mn.append(jnp.min(jnp.where(inb, sr, jnp.inf), axis=1, keepdims=True))
    pv_row = _pv_from(vlr_ref[...], vbr, mx, mn)
    vis_row = pv_row >= 0.5
    pv_blk = _pv_from(vl_ref[...], vb_ref[...], mx, mn)
    vis_blk = pv_blk >= 0.5

    nsq = jnp.sum(vfull * vfull, axis=1, keepdims=True)
    inv = 1.0 / (jnp.sqrt(nsq) + 1e-12)
    vfn = vfull * inv

    qsq = jnp.sum(vf * vf, axis=1, keepdims=True)
    qn = vf * (1.0 / (jnp.sqrt(qsq) + 1e-12))

    dn = (((1,), (1,)), ((), ()))
    S = jax.lax.dot_general(vf, vfull, dn, precision=jax.lax.Precision.HIGHEST,
                            preferred_element_type=jnp.float32)
    C = jax.lax.dot_general(qn, vfn, dn,
                            preferred_element_type=jnp.float32)

    mask = (vb_ref[...] == vbr) & vis_row
    C = jnp.where(mask, C, _NEG)

    iota = jax.lax.broadcasted_iota(jnp.int32, (rows, v), 1)

    for _ in range(_K):
        m = jnp.max(C, axis=1, keepdims=True)
        idx = jnp.min(jnp.where(C == m, iota, v), axis=1, keepdims=True)
        C = jnp.where(iota == idx, -jnp.inf, C)

    W = jnp.where(C == -jnp.inf, S, 0.0)
    den = jnp.sum(W, axis=1, keepdims=True)
    num0 = jnp.sum(W * flowT_ref[0:1, :], axis=1, keepdims=True)
    num1 = jnp.sum(W * flowT_ref[1:2, :], axis=1, keepdims=True)
    num2 = jnp.sum(W * flowT_ref[2:3, :], axis=1, keepdims=True)

    flow = flow_ref[...]
    out_ref[:, 0:1] = jnp.where(vis_blk, flow[:, 0:1], num0 / den)
    out_ref[:, 1:2] = jnp.where(vis_blk, flow[:, 1:2], num1 / den)
    out_ref[:, 2:3] = jnp.where(vis_blk, flow[:, 2:3], num2 / den)
    out_ref[:, 3:4] = pv_blk


@functools.partial(jax.jit, static_argnames=())
def _run(vtx, pts, vtx_feature, pts_feature, vis_logits, vtx_batch, pts_batch):
    V, D = vtx_feature.shape
    P = pts_feature.shape[0]
    nblk = V // _VBLK

    vb_col = vtx_batch[:, None]
    pb_row = pts_batch[None, :]
    vb_row = vtx_batch[None, :]
    vl_row = vis_logits.T
    ptsT = pts.T

    blk = lambda i: (i, 0)
    full = lambda i: (0, 0)
    params = pltpu.CompilerParams(dimension_semantics=("arbitrary",))

    flow = pl.pallas_call(
        _knn1_body,
        grid=(nblk,),
        in_specs=[
            pl.BlockSpec((_VBLK, D), blk),
            pl.BlockSpec((P, D), full),
            pl.BlockSpec((3, P), full),
            pl.BlockSpec((_VBLK, 1), blk),
            pl.BlockSpec((1, P), full),
            pl.BlockSpec((_VBLK, 3), blk),
        ],
        out_specs=pl.BlockSpec((_VBLK, 3), blk),
        out_shape=jax.ShapeDtypeStruct((V, 3), jnp.float32),
        compiler_params=params,
    )(vtx_feature, pts_feature, ptsT, vb_col, pb_row, vtx)

    out = pl.pallas_call(
        _knn2_body,
        grid=(nblk,),
        in_specs=[
            pl.BlockSpec((_VBLK, D), blk),
            pl.BlockSpec((V, D), full),
            pl.BlockSpec((_VBLK, 1), blk),
            pl.BlockSpec((1, V), full),
            pl.BlockSpec((_VBLK, 1), blk),
            pl.BlockSpec((1, V), full),
            pl.BlockSpec((_VBLK, 3), blk),
            pl.BlockSpec((3, V), full),
        ],
        out_specs=pl.BlockSpec((_VBLK, 4), blk),
        out_shape=jax.ShapeDtypeStruct((V, 4), jnp.float32),
        compiler_params=params,
    )(vtx_feature, vtx_feature, vb_col, vb_row, vis_logits, vl_row,
      flow, flow.T)

    return out


def kernel(vtx, pts, vtx_feature, pts_feature, vis_logits, vtx_batch, pts_batch):
    return _run(vtx, pts, vtx_feature, pts_feature, vis_logits,
                vtx_batch, pts_batch)


# speedup vs baseline: 1.5550x; 1.5550x over previous
import functools

import jax
import jax.numpy as jnp
import numpy as np
from jax.experimental import pallas as pl
from jax.experimental.pallas import tpu as pltpu
from jax.sharding import PartitionSpec as P

_K = 8
_NB = 4
_NEG = -1e30
_VBLK = 256


def _knn1_body(vf_ref, pf_ref, ptsT_ref, vb_ref, pb_ref, vtx_ref, out_ref):
    vf = vf_ref[...]
    pf = pf_ref[...]
    rows, p = vf.shape[0], pf.shape[0]

    nsq = jnp.sum(pf * pf, axis=1, keepdims=True)
    inv = 1.0 / (jnp.sqrt(nsq) + 1e-12)
    pfn = pf * inv

    qsq = jnp.sum(vf * vf, axis=1, keepdims=True)
    qn = vf * (1.0 / (jnp.sqrt(qsq) + 1e-12))

    dn = (((1,), (1,)), ((), ()))
    S = jax.lax.dot_general(vf, pf, dn, precision=jax.lax.Precision.HIGHEST,
                            preferred_element_type=jnp.float32)
    C = jax.lax.dot_general(qn, pfn, dn,
                            preferred_element_type=jnp.float32)

    mask = vb_ref[...] == pb_ref[...]
    C = jnp.where(mask, C, _NEG)

    iota = jax.lax.broadcasted_iota(jnp.int32, (rows, p), 1)
    vtx = vtx_ref[...]

    for _ in range(_K):
        m = jnp.max(C, axis=1, keepdims=True)
        idx = jnp.min(jnp.where(C == m, iota, p), axis=1, keepdims=True)
        C = jnp.where(iota == idx, -jnp.inf, C)

    W = jnp.where(C == -jnp.inf, S, 0.0)
    den = jnp.sum(W, axis=1, keepdims=True)
    num0 = jnp.sum(W * ptsT_ref[0:1, :], axis=1, keepdims=True)
    num1 = jnp.sum(W * ptsT_ref[1:2, :], axis=1, keepdims=True)
    num2 = jnp.sum(W * ptsT_ref[2:3, :], axis=1, keepdims=True)

    out_ref[:, 0:1] = num0 / den - vtx[:, 0:1]
    out_ref[:, 1:2] = num1 / den - vtx[:, 1:2]
    out_ref[:, 2:3] = num2 / den - vtx[:, 2:3]


def _pv_from(logits, batch, mx, mn):
    s = jax.nn.sigmoid(logits)
    out = jnp.zeros_like(s)
    for b in range(_NB):
        out = jnp.where(batch == b, (s - mn[b]) / (mx[b] - mn[b]), out)
    return out


def _knn2_body(vf_ref, vfull_ref, vb_ref, vbr_ref, vl_ref, vlr_ref,
               flow_ref, flowT_ref, out_ref):
    vf = vf_ref[...]
    vfull = vfull_ref[...]
    rows, v = vf.shape[0], vfull.shape[0]

    vbr = vbr_ref[...]
    sr = jax.nn.sigmoid(vlr_ref[...])
    mx, mn = [], []
    for b in range(_NB):
        inb = vbr == b
        mx.append(jnp.max(jnp.where(inb, sr, -jnp.inf), axis=1, keepdims=True))
        mn.append(jnp.min(jnp.where(inb, sr, jnp.inf), axis=1, keepdims=True))
    pv_row = _pv_from(vlr_ref[...], vbr, mx, mn)
    vis_row = pv_row >= 0.5
    pv_blk = _pv_from(vl_ref[...], vb_ref[...], mx, mn)
    vis_blk = pv_blk >= 0.5

    nsq = jnp.sum(vfull * vfull, axis=1, keepdims=True)
    inv = 1.0 / (jnp.sqrt(nsq) + 1e-12)
    vfn = vfull * inv

    qsq = jnp.sum(vf * vf, axis=1, keepdims=True)
    qn = vf * (1.0 / (jnp.sqrt(qsq) + 1e-12))

    dn = (((1,), (1,)), ((), ()))
    S = jax.lax.dot_general(vf, vfull, dn, precision=jax.lax.Precision.HIGHEST,
                            preferred_element_type=jnp.float32)
    C = jax.lax.dot_general(qn, vfn, dn,
                            preferred_element_type=jnp.float32)

    mask = (vb_ref[...] == vbr) & vis_row
    C = jnp.where(mask, C, _NEG)

    iota = jax.lax.broadcasted_iota(jnp.int32, (rows, v), 1)

    for _ in range(_K):
        m = jnp.max(C, axis=1, keepdims=True)
        idx = jnp.min(jnp.where(C == m, iota, v), axis=1, keepdims=True)
        C = jnp.where(iota == idx, -jnp.inf, C)

    W = jnp.where(C == -jnp.inf, S, 0.0)
    den = jnp.sum(W, axis=1, keepdims=True)
    num0 = jnp.sum(W * flowT_ref[0:1, :], axis=1, keepdims=True)
    num1 = jnp.sum(W * flowT_ref[1:2, :], axis=1, keepdims=True)
    num2 = jnp.sum(W * flowT_ref[2:3, :], axis=1, keepdims=True)

    flow = flow_ref[...]
    out_ref[:, 0:1] = jnp.where(vis_blk, flow[:, 0:1], num0 / den)
    out_ref[:, 1:2] = jnp.where(vis_blk, flow[:, 1:2], num1 / den)
    out_ref[:, 2:3] = jnp.where(vis_blk, flow[:, 2:3], num2 / den)
    out_ref[:, 3:4] = pv_blk


_PARAMS = pltpu.CompilerParams(dimension_semantics=("arbitrary",))
_BLK = lambda i: (i, 0)
_FULL = lambda i: (0, 0)


def _phase1(vf_loc, pf, ptsT, vb_col, pb_row, vtx_loc):
    vloc, d = vf_loc.shape
    p = pf.shape[0]
    nblk = vloc // _VBLK
    return pl.pallas_call(
        _knn1_body,
        grid=(nblk,),
        in_specs=[
            pl.BlockSpec((_VBLK, d), _BLK),
            pl.BlockSpec((p, d), _FULL),
            pl.BlockSpec((3, p), _FULL),
            pl.BlockSpec((_VBLK, 1), _BLK),
            pl.BlockSpec((1, p), _FULL),
            pl.BlockSpec((_VBLK, 3), _BLK),
        ],
        out_specs=pl.BlockSpec((_VBLK, 3), _BLK),
        out_shape=jax.ShapeDtypeStruct((vloc, 3), jnp.float32),
        compiler_params=_PARAMS,
    )(vf_loc, pf, ptsT, vb_col, pb_row, vtx_loc)


def _phase2(vf_loc, vf_full, vb_col, vb_row, vl_col, vl_row, flow_loc, flowT):
    vloc, d = vf_loc.shape
    v = vf_full.shape[0]
    nblk = vloc // _VBLK
    return pl.pallas_call(
        _knn2_body,
        grid=(nblk,),
        in_specs=[
            pl.BlockSpec((_VBLK, d), _BLK),
            pl.BlockSpec((v, d), _FULL),
            pl.BlockSpec((_VBLK, 1), _BLK),
            pl.BlockSpec((1, v), _FULL),
            pl.BlockSpec((_VBLK, 1), _BLK),
            pl.BlockSpec((1, v), _FULL),
            pl.BlockSpec((_VBLK, 3), _BLK),
            pl.BlockSpec((3, v), _FULL),
        ],
        out_specs=pl.BlockSpec((_VBLK, 4), _BLK),
        out_shape=jax.ShapeDtypeStruct((vloc, 4), jnp.float32),
        compiler_params=_PARAMS,
    )(vf_loc, vf_full, vb_col, vb_row, vl_col, vl_row, flow_loc, flowT)


def _run_single(vtx, pts, vf, pf, vl, vb, pb):
    flow = _phase1(vf, pf, pts.T, vb[:, None], pb[None, :], vtx)
    return _phase2(vf, vf, vb[:, None], vb[None, :], vl, vl.T, flow, flow.T)


def _build(n_dev):
    if n_dev < 2:
        return jax.jit(_run_single)

    mesh = jax.sharding.Mesh(np.array(jax.devices()[:2]), ("x",))

    def _body(vtx, pts, vf, pf, vl, vb, pb):
        i = jax.lax.axis_index("x")
        half = vf.shape[0] // 2
        sl = lambda a: jax.lax.dynamic_slice_in_dim(a, i * half, half, axis=0)
        vf_h, vtx_h, vl_h, vb_h = sl(vf), sl(vtx), sl(vl), sl(vb)
        flow_h = _phase1(vf_h, pf, pts.T, vb_h[:, None], pb[None, :], vtx_h)
        flow_f = jax.lax.all_gather(flow_h, "x", axis=0, tiled=True)
        return _phase2(vf_h, vf, vb_h[:, None], vb[None, :], vl_h, vl.T,
                       flow_h, flow_f.T)

    try:
        from jax.experimental.shard_map import shard_map
    except ImportError:
        shard_map = jax.shard_map

    sharded = shard_map(
        _body, mesh=mesh,
        in_specs=(P(), P(), P(), P(), P(), P(), P()),
        out_specs=P("x"),
        check_rep=False,
    )
    return jax.jit(sharded)


_CACHE = {}


def kernel(vtx, pts, vtx_feature, pts_feature, vis_logits, vtx_batch, pts_batch):
    n_dev = len(jax.devices())
    fn = _CACHE.get(n_dev)
    if fn is None:
        fn = _CACHE[n_dev] = _build(n_dev)
    return fn(vtx, pts, vtx_feature, pts_feature, vis_logits,
              vtx_batch, pts_batch)
